# baseline (speedup 1.0000x reference)
import functools

import jax
import jax.numpy as jnp
from jax import lax
from jax.experimental import pallas as pl
from jax.experimental.pallas import tpu as pltpu
from jax.experimental.pallas import tpu_sc as plsc


def _fs_body(M, xr_ref, cf_ref, wf_ref, rt5_ref, fs_ref):
    DM = xr_ref.shape[1]
    R = rt5_ref.shape[1]
    xr = xr_ref[...]
    cf = cf_ref[0:1, :]
    wf = wf_ref[0:1, :]
    g = -((xr - cf) ** 2) / (2.0 * wf * wf) + 1e-9

    mm = lax.broadcasted_iota(jnp.int32, (DM, R), 0) % M
    ohf = (rt5_ref[...] == mm).astype(jnp.float32)

    y = jnp.floor(jnp.maximum(g, -120.0) * 262144.0 + 0.5)
    hi = jnp.floor(y * (1.0 / 8192.0))
    lo = y - hi * 8192.0
    s_hi = lax.dot_general(hi, ohf, (((1,), (0,)), ((), ())),
                           preferred_element_type=jnp.float32)
    s_lo = lax.dot_general(lo, ohf, (((1,), (0,)), ((), ())),
                           preferred_element_type=jnp.float32)
    s = s_hi * (1.0 / 32.0) + s_lo * (1.0 / 262144.0)
    fs_ref[...] = jnp.exp(s)


def _firing_strengths(x, centers, widths, rules):
    B, D = x.shape
    M = centers.shape[1]
    R = rules.shape[0]
    DM = D * M
    BB = 256
    xr = jnp.repeat(x, M, axis=1)
    cf = jnp.broadcast_to(centers.reshape(1, DM), (8, DM))
    wf = jnp.broadcast_to(widths.reshape(1, DM), (8, DM))
    rt5 = jnp.repeat(rules.T, M, axis=0)
    return pl.pallas_call(
        functools.partial(_fs_body, M),
        grid=(B // BB,),
        in_specs=[
            pl.BlockSpec((BB, DM), lambda i: (i, 0)),
            pl.BlockSpec((8, DM), lambda i: (0, 0)),
            pl.BlockSpec((8, DM), lambda i: (0, 0)),
            pl.BlockSpec((DM, R), lambda i: (0, 0)),
        ],
        out_specs=pl.BlockSpec((BB, R), lambda i: (i, 0)),
        out_shape=jax.ShapeDtypeStruct((B, R), jnp.float32),
    )(xr, cf, wf, rt5)


def _sc_topk_mask(fs, K):
    B, R = fs.shape
    info = plsc.get_sparse_core_info()
    NC, NS, L = info.num_cores, info.num_subcores, info.num_lanes
    NW = NC * NS
    rows_per_w = B // NW
    CH = min(8, rows_per_w)
    NSL = R // L

    mesh = plsc.VectorSubcoreMesh(core_axis_name="c", subcore_axis_name="s")

    @functools.partial(
        pl.kernel,
        out_type=(
            jax.ShapeDtypeStruct((B, R), jnp.float32),
            jax.ShapeDtypeStruct((B, R), jnp.float32),
        ),
        mesh=mesh,
        scratch_types=[
            pltpu.VMEM((CH, R), jnp.float32),
            pltpu.VMEM((CH, R), jnp.float32),
            pltpu.VMEM((L,), jnp.int32),
            pltpu.VMEM((L,), jnp.int32),
        ],
    )
    def k(fs_hbm, fir_hbm, msk_hbm, buf, mbuf, aref, bref):
        wid = lax.axis_index("s") * NC + lax.axis_index("c")
        base = wid * rows_per_w
        io = lax.iota(jnp.int32, L)
        one = jnp.full((L,), 1, jnp.int32)
        zero = jnp.zeros((L,), jnp.int32)

        def _splat_min(v):
            for sh in (8, 4, 2, 1):
                v = jnp.minimum(v, v.at[(io + sh) & (L - 1)].get(
                    mode="promise_in_bounds"))
            return v

        def _splat_max(v):
            for sh in (8, 4, 2, 1):
                v = jnp.maximum(v, v.at[(io + sh) & (L - 1)].get(
                    mode="promise_in_bounds"))
            return v

        def _splat_sum(v):
            for sh in (8, 4, 2, 1):
                v = v + v.at[(io + sh) & (L - 1)].get(
                    mode="promise_in_bounds")
            return v

        def row_body(r, _):
            aref[...] = jnp.full((L,), 0x7F7FFFFF, jnp.int32)
            bref[...] = zero

            def mm_body(j, _):
                b = lax.bitcast_convert_type(
                    buf[r, pl.ds(j * L, L)], jnp.int32)
                aref[...] = jnp.minimum(aref[...], b)
                bref[...] = jnp.maximum(bref[...], b)
                return 0
            lax.fori_loop(0, NSL, mm_body, 0)
            lo0 = _splat_min(aref[...])[0]
            hi0 = _splat_max(bref[...])[0]

            def bs_body(i, c):
                lo, hi = c
                mid = lo + ((hi - lo + 1) >> 1)
                aref[...] = zero

                def cbody(j, _):
                    b = lax.bitcast_convert_type(
                        buf[r, pl.ds(j * L, L)], jnp.int32)
                    aref[...] = aref[...] + jnp.where(b >= mid, one, zero)
                    return 0
                lax.fori_loop(0, jnp.where(lo < hi, NSL, 0), cbody, 0)
                cnt = _splat_sum(aref[...])[0]
                ge = cnt >= K
                new_lo = jnp.where(jnp.logical_and(lo < hi, ge), mid, lo)
                new_hi = jnp.where(jnp.logical_and(lo < hi, jnp.logical_not(ge)),
                                   mid - 1, hi)
                return (new_lo, new_hi)

            t, _ = lax.fori_loop(0, 31, bs_body, (lo0, hi0))

            aref[...] = zero

            def cg_body(j, _):
                b = lax.bitcast_convert_type(
                    buf[r, pl.ds(j * L, L)], jnp.int32)
                aref[...] = aref[...] + jnp.where(b > t, one, zero)
                return 0
            lax.fori_loop(0, NSL, cg_body, 0)
            navail = K - _splat_sum(aref[...])[0]

            bref[...] = zero

            def mk_body(j, _):
                v = buf[r, pl.ds(j * L, L)]
                b = lax.bitcast_convert_type(v, jnp.int32)
                eqi = jnp.where(b == t, one, zero)
                cs = eqi
                for sh in (1, 2, 4, 8):
                    g = cs.at[(io - sh) & (L - 1)].get(
                        mode="promise_in_bounds")
                    cs = cs + jnp.where(io >= sh, g, zero)
                pre = cs - eqi + bref[...]
                mf = jnp.where(
                    b > t,
                    jnp.full((L,), 1.0, jnp.float32),
                    jnp.where(
                        jnp.logical_and(b == t, pre < navail),
                        jnp.full((L,), 1.0, jnp.float32),
                        jnp.zeros((L,), jnp.float32)))
                mbuf[r, pl.ds(j * L, L)] = mf
                buf[r, pl.ds(j * L, L)] = v * mf
                bref[...] = bref[...] + _splat_sum(eqi)
                return 0

            lax.fori_loop(0, NSL, mk_body, 0)
            return 0

        for ci in range(rows_per_w // CH):
            row0 = base + ci * CH
            pltpu.sync_copy(fs_hbm.at[pl.ds(row0, CH)], buf)
            lax.fori_loop(0, CH, row_body, 0)
            pltpu.sync_copy(mbuf, msk_hbm.at[pl.ds(row0, CH)])
            pltpu.sync_copy(buf, fir_hbm.at[pl.ds(row0, CH)])

    return k(fs)


def _combine_body(C, fir_ref, xe_ref, cons_ref, out_ref, nrm_ref):
    fir = fir_ref[...]
    denom = jnp.sum(fir, axis=1, keepdims=True) + 1e-9
    nrm = fir / denom
    nrm_ref[...] = nrm
    cons2 = cons_ref[...]
    nj = cons2.shape[1] // C
    csum = cons2[:, 0:C]
    for j in range(1, nj):
        csum = csum + cons2[:, j * C:(j + 1) * C]
    w = lax.dot_general(nrm, csum, (((1,), (0,)), ((), ())),
                        preferred_element_type=jnp.float32)
    sx = jnp.sum(xe_ref[...], axis=1, keepdims=True)
    out_ref[...] = sx * w


def _combine(fir, x, consequents):
    B, R = fir.shape
    D = x.shape[1]
    C = consequents.shape[2]
    xe = jnp.concatenate([x, jnp.ones((B, 1), x.dtype)], axis=1)
    cons = consequents.reshape(R, (D + 1) * C)
    return pl.pallas_call(
        functools.partial(_combine_body, C),
        in_specs=[
            pl.BlockSpec((B, R), lambda: (0, 0)),
            pl.BlockSpec((B, D + 1), lambda: (0, 0)),
            pl.BlockSpec((R, (D + 1) * C), lambda: (0, 0)),
        ],
        out_specs=[
            pl.BlockSpec((B, C), lambda: (0, 0)),
            pl.BlockSpec((B, R), lambda: (0, 0)),
        ],
        out_shape=[
            jax.ShapeDtypeStruct((B, C), jnp.float32),
            jax.ShapeDtypeStruct((B, R), jnp.float32),
        ],
    )(fir, xe, cons)


def kernel(x, centers, widths, consequents, rules):
    R = rules.shape[0]
    K = max(1, int(0.2 * R))
    fs = _firing_strengths(x, centers, widths, rules)
    firing, mask = _sc_topk_mask(fs, K)
    rule_outputs, normalized = _combine(firing, x, consequents)
    return (rule_outputs, normalized, mask)


# speedup vs baseline: 1.7857x; 1.7857x over previous
import functools

import jax
import jax.numpy as jnp
from jax import lax
from jax.experimental import pallas as pl
from jax.experimental.pallas import tpu as pltpu
from jax.experimental.pallas import tpu_sc as plsc


def _fs_body(M, xr_ref, cf_ref, wf_ref, rt5_ref, fs_ref):
    DM = xr_ref.shape[1]
    R = rt5_ref.shape[1]
    xr = xr_ref[...]
    cf = cf_ref[0:1, :]
    wf = wf_ref[0:1, :]
    g = -((xr - cf) ** 2) / (2.0 * wf * wf) + 1e-9

    mm = lax.broadcasted_iota(jnp.int32, (DM, R), 0) % M
    ohf = (rt5_ref[...] == mm).astype(jnp.float32)

    y = jnp.floor(jnp.maximum(g, -120.0) * 262144.0 + 0.5)
    hi = jnp.floor(y * (1.0 / 8192.0))
    lo = y - hi * 8192.0
    s_hi = lax.dot_general(hi, ohf, (((1,), (0,)), ((), ())),
                           preferred_element_type=jnp.float32)
    s_lo = lax.dot_general(lo, ohf, (((1,), (0,)), ((), ())),
                           preferred_element_type=jnp.float32)
    s = s_hi * (1.0 / 32.0) + s_lo * (1.0 / 262144.0)
    fs_ref[...] = jnp.exp(s)


def _firing_strengths(x, centers, widths, rules):
    B, D = x.shape
    M = centers.shape[1]
    R = rules.shape[0]
    DM = D * M
    BB = 256
    xr = jnp.repeat(x, M, axis=1)
    cf = jnp.broadcast_to(centers.reshape(1, DM), (8, DM))
    wf = jnp.broadcast_to(widths.reshape(1, DM), (8, DM))
    rt5 = jnp.repeat(rules.T, M, axis=0)
    return pl.pallas_call(
        functools.partial(_fs_body, M),
        grid=(B // BB,),
        in_specs=[
            pl.BlockSpec((BB, DM), lambda i: (i, 0)),
            pl.BlockSpec((8, DM), lambda i: (0, 0)),
            pl.BlockSpec((8, DM), lambda i: (0, 0)),
            pl.BlockSpec((DM, R), lambda i: (0, 0)),
        ],
        out_specs=pl.BlockSpec((BB, R), lambda i: (i, 0)),
        out_shape=jax.ShapeDtypeStruct((B, R), jnp.float32),
    )(xr, cf, wf, rt5)


def _sc_topk_mask(fs, K):
    B, R = fs.shape
    info = plsc.get_sparse_core_info()
    NC, NS, L = info.num_cores, info.num_subcores, info.num_lanes
    NW = NC * NS
    rows_per_w = B // NW
    CH = min(16, rows_per_w)
    NSL = R // L

    mesh = plsc.VectorSubcoreMesh(core_axis_name="c", subcore_axis_name="s")

    @functools.partial(
        pl.kernel,
        out_type=(
            jax.ShapeDtypeStruct((B, R), jnp.float32),
            jax.ShapeDtypeStruct((B, R), jnp.float32),
        ),
        mesh=mesh,
        scratch_types=[
            pltpu.VMEM((CH, R), jnp.float32),
            pltpu.VMEM((CH, R), jnp.float32),
            pltpu.VMEM((L,), jnp.int32),
            pltpu.VMEM((L,), jnp.int32),
        ],
    )
    def k(fs_hbm, fir_hbm, msk_hbm, buf, mbuf, aref, bref):
        wid = lax.axis_index("s") * NC + lax.axis_index("c")
        base = wid * rows_per_w
        io = lax.iota(jnp.int32, L)
        one = jnp.full((L,), 1, jnp.int32)
        zero = jnp.zeros((L,), jnp.int32)

        def _splat_min(v):
            for sh in (8, 4, 2, 1):
                v = jnp.minimum(v, v.at[(io + sh) & (L - 1)].get(
                    mode="promise_in_bounds"))
            return v

        def _splat_max(v):
            for sh in (8, 4, 2, 1):
                v = jnp.maximum(v, v.at[(io + sh) & (L - 1)].get(
                    mode="promise_in_bounds"))
            return v

        def _splat_sum(v):
            for sh in (8, 4, 2, 1):
                v = v + v.at[(io + sh) & (L - 1)].get(
                    mode="promise_in_bounds")
            return v

        def row_body(r, _):
            aref[...] = jnp.full((L,), 0x7F7FFFFF, jnp.int32)
            bref[...] = zero

            def mm_body(j, _):
                b = lax.bitcast_convert_type(
                    buf[r, pl.ds(j * L, L)], jnp.int32)
                aref[...] = jnp.minimum(aref[...], b)
                bref[...] = jnp.maximum(bref[...], b)
                return 0
            lax.fori_loop(0, NSL, mm_body, 0)
            lo0 = _splat_min(aref[...])[0]
            hi0 = _splat_max(bref[...])[0]

            def bs_body(i, c):
                lo, hi = c
                mid = lo + ((hi - lo + 1) >> 1)
                aref[...] = zero

                def cbody(j, _):
                    b = lax.bitcast_convert_type(
                        buf[r, pl.ds(j * L, L)], jnp.int32)
                    aref[...] = aref[...] + jnp.where(b >= mid, one, zero)
                    return 0
                lax.fori_loop(0, jnp.where(lo < hi, NSL, 0), cbody, 0)
                cnt = _splat_sum(aref[...])[0]
                ge = cnt >= K
                new_lo = jnp.where(jnp.logical_and(lo < hi, ge), mid, lo)
                new_hi = jnp.where(jnp.logical_and(lo < hi, jnp.logical_not(ge)),
                                   mid - 1, hi)
                return (new_lo, new_hi)

            t, _ = lax.fori_loop(0, 31, bs_body, (lo0, hi0))

            tie = lo0 == hi0
            n_slow = jnp.where(tie, 0, NSL)
            n_fast = jnp.where(tie, NSL, 0)

            aref[...] = zero

            def cg_body(j, _):
                b = lax.bitcast_convert_type(
                    buf[r, pl.ds(j * L, L)], jnp.int32)
                aref[...] = aref[...] + jnp.where(b > t, one, zero)
                return 0
            lax.fori_loop(0, n_slow, cg_body, 0)
            navail = K - _splat_sum(aref[...])[0]

            bref[...] = zero

            def mk_body(j, _):
                v = buf[r, pl.ds(j * L, L)]
                b = lax.bitcast_convert_type(v, jnp.int32)
                eqi = jnp.where(b == t, one, zero)
                cs = eqi
                for sh in (1, 2, 4, 8):
                    g = cs.at[(io - sh) & (L - 1)].get(
                        mode="promise_in_bounds")
                    cs = cs + jnp.where(io >= sh, g, zero)
                pre = cs - eqi + bref[...]
                mf = jnp.where(
                    b > t,
                    jnp.full((L,), 1.0, jnp.float32),
                    jnp.where(
                        jnp.logical_and(b == t, pre < navail),
                        jnp.full((L,), 1.0, jnp.float32),
                        jnp.zeros((L,), jnp.float32)))
                mbuf[r, pl.ds(j * L, L)] = mf
                buf[r, pl.ds(j * L, L)] = v * mf
                bref[...] = bref[...] + _splat_sum(eqi)
                return 0

            lax.fori_loop(0, n_slow, mk_body, 0)

            def mkf_body(j, _):
                v = buf[r, pl.ds(j * L, L)]
                mf = jnp.where(j * L + io < K,
                               jnp.full((L,), 1.0, jnp.float32),
                               jnp.zeros((L,), jnp.float32))
                mbuf[r, pl.ds(j * L, L)] = mf
                buf[r, pl.ds(j * L, L)] = v * mf
                return 0

            lax.fori_loop(0, n_fast, mkf_body, 0)
            return 0

        for ci in range(rows_per_w // CH):
            row0 = base + ci * CH
            pltpu.sync_copy(fs_hbm.at[pl.ds(row0, CH)], buf)
            lax.fori_loop(0, CH, row_body, 0)
            pltpu.sync_copy(mbuf, msk_hbm.at[pl.ds(row0, CH)])
            pltpu.sync_copy(buf, fir_hbm.at[pl.ds(row0, CH)])

    return k(fs)


def _combine_body(C, fir_ref, xe_ref, cons_ref, out_ref, nrm_ref):
    fir = fir_ref[...]
    denom = jnp.sum(fir, axis=1, keepdims=True) + 1e-9
    nrm = fir / denom
    nrm_ref[...] = nrm
    cons2 = cons_ref[...]
    nj = cons2.shape[1] // C
    csum = cons2[:, 0:C]
    for j in range(1, nj):
        csum = csum + cons2[:, j * C:(j + 1) * C]
    w = lax.dot_general(nrm, csum, (((1,), (0,)), ((), ())),
                        preferred_element_type=jnp.float32)
    sx = jnp.sum(xe_ref[...], axis=1, keepdims=True)
    out_ref[...] = sx * w


def _combine(fir, x, consequents):
    B, R = fir.shape
    D = x.shape[1]
    C = consequents.shape[2]
    xe = jnp.concatenate([x, jnp.ones((B, 1), x.dtype)], axis=1)
    cons = consequents.reshape(R, (D + 1) * C)
    return pl.pallas_call(
        functools.partial(_combine_body, C),
        in_specs=[
            pl.BlockSpec((B, R), lambda: (0, 0)),
            pl.BlockSpec((B, D + 1), lambda: (0, 0)),
            pl.BlockSpec((R, (D + 1) * C), lambda: (0, 0)),
        ],
        out_specs=[
            pl.BlockSpec((B, C), lambda: (0, 0)),
            pl.BlockSpec((B, R), lambda: (0, 0)),
        ],
        out_shape=[
            jax.ShapeDtypeStruct((B, C), jnp.float32),
            jax.ShapeDtypeStruct((B, R), jnp.float32),
        ],
    )(fir, xe, cons)


def kernel(x, centers, widths, consequents, rules):
    R = rules.shape[0]
    K = max(1, int(0.2 * R))
    fs = _firing_strengths(x, centers, widths, rules)
    firing, mask = _sc_topk_mask(fs, K)
    rule_outputs, normalized = _combine(firing, x, consequents)
    return (rule_outputs, normalized, mask)


# speedup vs baseline: 1.9286x; 1.0800x over previous
import functools

import jax
import jax.numpy as jnp
from jax import lax
from jax.experimental import pallas as pl
from jax.experimental.pallas import tpu as pltpu
from jax.experimental.pallas import tpu_sc as plsc


def _fs_body(M, xr_ref, cf_ref, wf_ref, rt5_ref, fs_ref):
    DM = xr_ref.shape[1]
    R = rt5_ref.shape[1]
    xr = xr_ref[...]
    cf = cf_ref[0:1, :]
    wf = wf_ref[0:1, :]
    g = -((xr - cf) ** 2) / (2.0 * wf * wf) + 1e-9

    mm = lax.broadcasted_iota(jnp.int32, (DM, R), 0) % M
    ohf = (rt5_ref[...] == mm).astype(jnp.float32)

    y = jnp.floor(jnp.maximum(g, -120.0) * 262144.0 + 0.5)
    hi = jnp.floor(y * (1.0 / 8192.0))
    lo = y - hi * 8192.0
    s_hi = lax.dot_general(hi, ohf, (((1,), (0,)), ((), ())),
                           preferred_element_type=jnp.float32)
    s_lo = lax.dot_general(lo, ohf, (((1,), (0,)), ((), ())),
                           preferred_element_type=jnp.float32)
    s = s_hi * (1.0 / 32.0) + s_lo * (1.0 / 262144.0)
    fs_ref[...] = jnp.exp(s)


def _firing_strengths(x, centers, widths, rules):
    B, D = x.shape
    M = centers.shape[1]
    R = rules.shape[0]
    DM = D * M
    BB = 256
    xr = jnp.repeat(x, M, axis=1)
    cf = jnp.broadcast_to(centers.reshape(1, DM), (8, DM))
    wf = jnp.broadcast_to(widths.reshape(1, DM), (8, DM))
    rt5 = jnp.repeat(rules.T, M, axis=0)
    return pl.pallas_call(
        functools.partial(_fs_body, M),
        grid=(B // BB,),
        in_specs=[
            pl.BlockSpec((BB, DM), lambda i: (i, 0)),
            pl.BlockSpec((8, DM), lambda i: (0, 0)),
            pl.BlockSpec((8, DM), lambda i: (0, 0)),
            pl.BlockSpec((DM, R), lambda i: (0, 0)),
        ],
        out_specs=pl.BlockSpec((BB, R), lambda i: (i, 0)),
        out_shape=jax.ShapeDtypeStruct((B, R), jnp.float32),
    )(xr, cf, wf, rt5)


def _sc_topk_mask(fs, K):
    B, R = fs.shape
    info = plsc.get_sparse_core_info()
    NC, NS, L = info.num_cores, info.num_subcores, info.num_lanes
    NW = NC * NS
    rows_per_w = B // NW
    CH = min(16, rows_per_w)
    NSL = R // L

    mesh = plsc.VectorSubcoreMesh(core_axis_name="c", subcore_axis_name="s")

    @functools.partial(
        pl.kernel,
        out_type=(
            jax.ShapeDtypeStruct((B, R), jnp.float32),
            jax.ShapeDtypeStruct((B, R), jnp.float32),
        ),
        mesh=mesh,
        scratch_types=[
            pltpu.VMEM((CH, R), jnp.float32),
            pltpu.VMEM((CH, R), jnp.float32),
            pltpu.VMEM((L,), jnp.int32),
            pltpu.VMEM((L,), jnp.int32),
        ],
    )
    def k(fs_hbm, fir_hbm, msk_hbm, buf, mbuf, aref, bref):
        wid = lax.axis_index("s") * NC + lax.axis_index("c")
        base = wid * rows_per_w
        io = lax.iota(jnp.int32, L)
        one = jnp.full((L,), 1, jnp.int32)
        zero = jnp.zeros((L,), jnp.int32)

        def _splat_min(v):
            for sh in (8, 4, 2, 1):
                v = jnp.minimum(v, v.at[(io + sh) & (L - 1)].get(
                    mode="promise_in_bounds"))
            return v

        def _splat_max(v):
            for sh in (8, 4, 2, 1):
                v = jnp.maximum(v, v.at[(io + sh) & (L - 1)].get(
                    mode="promise_in_bounds"))
            return v

        def _splat_sum(v):
            for sh in (8, 4, 2, 1):
                v = v + v.at[(io + sh) & (L - 1)].get(
                    mode="promise_in_bounds")
            return v

        def row_body(r, _):
            def mm_body(j, c):
                mn, mx = c
                b = lax.bitcast_convert_type(
                    buf[r, pl.ds(j * L, L)], jnp.int32)
                return jnp.minimum(mn, b), jnp.maximum(mx, b)
            mnv, mxv = lax.fori_loop(
                0, NSL, mm_body,
                (jnp.full((L,), 0x7F7FFFFF, jnp.int32), zero))
            lo0 = _splat_min(mnv)[0]
            hi0 = _splat_max(mxv)[0]

            def bs_body(i, c):
                lo, hi = c
                mid = lo + ((hi - lo + 1) >> 1)
                aref[...] = zero

                def cbody(j, _):
                    b = lax.bitcast_convert_type(
                        buf[r, pl.ds(j * L, L)], jnp.int32)
                    aref[...] = aref[...] + jnp.where(b >= mid, one, zero)
                    return 0
                lax.fori_loop(0, jnp.where(lo < hi, NSL, 0), cbody, 0)
                cnt = _splat_sum(aref[...])[0]
                ge = cnt >= K
                new_lo = jnp.where(jnp.logical_and(lo < hi, ge), mid, lo)
                new_hi = jnp.where(jnp.logical_and(lo < hi, jnp.logical_not(ge)),
                                   mid - 1, hi)
                return (new_lo, new_hi)

            t, _ = lax.fori_loop(0, 31, bs_body, (lo0, hi0))

            tie = lo0 == hi0
            n_slow = jnp.where(tie, 0, NSL)
            n_fast = jnp.where(tie, NSL, 0)

            aref[...] = zero

            def cg_body(j, _):
                b = lax.bitcast_convert_type(
                    buf[r, pl.ds(j * L, L)], jnp.int32)
                aref[...] = aref[...] + jnp.where(b > t, one, zero)
                return 0
            lax.fori_loop(0, n_slow, cg_body, 0)
            navail = K - _splat_sum(aref[...])[0]

            bref[...] = zero

            def mk_body(j, _):
                v = buf[r, pl.ds(j * L, L)]
                b = lax.bitcast_convert_type(v, jnp.int32)
                eqi = jnp.where(b == t, one, zero)
                cs = eqi
                for sh in (1, 2, 4, 8):
                    g = cs.at[(io - sh) & (L - 1)].get(
                        mode="promise_in_bounds")
                    cs = cs + jnp.where(io >= sh, g, zero)
                pre = cs - eqi + bref[...]
                mf = jnp.where(
                    b > t,
                    jnp.full((L,), 1.0, jnp.float32),
                    jnp.where(
                        jnp.logical_and(b == t, pre < navail),
                        jnp.full((L,), 1.0, jnp.float32),
                        jnp.zeros((L,), jnp.float32)))
                mbuf[r, pl.ds(j * L, L)] = mf
                buf[r, pl.ds(j * L, L)] = v * mf
                bref[...] = bref[...] + _splat_sum(eqi)
                return 0

            lax.fori_loop(0, n_slow, mk_body, 0)

            KF = K // L
            REM = K % L
            ZS = KF + (1 if REM else 0)
            onesf = jnp.full((L,), 1.0, jnp.float32)
            zerof = jnp.zeros((L,), jnp.float32)

            def f1_body(j, _):
                mbuf[r, pl.ds(j * L, L)] = onesf
                return 0
            lax.fori_loop(0, jnp.where(tie, KF, 0), f1_body, 0)

            if REM:
                def fb_body(j, _):
                    v = buf[r, pl.ds(KF * L, L)]
                    mf = jnp.where(io < REM, onesf, zerof)
                    mbuf[r, pl.ds(KF * L, L)] = mf
                    buf[r, pl.ds(KF * L, L)] = v * mf
                    return 0
                lax.fori_loop(0, jnp.where(tie, 1, 0), fb_body, 0)

            def f3_body(j, _):
                mbuf[r, pl.ds(j * L, L)] = zerof
                buf[r, pl.ds(j * L, L)] = zerof
                return 0
            lax.fori_loop(ZS, jnp.where(tie, NSL, ZS), f3_body, 0)
            return 0

        for ci in range(rows_per_w // CH):
            row0 = base + ci * CH
            pltpu.sync_copy(fs_hbm.at[pl.ds(row0, CH)], buf)
            lax.fori_loop(0, CH, row_body, 0)
            pltpu.sync_copy(mbuf, msk_hbm.at[pl.ds(row0, CH)])
            pltpu.sync_copy(buf, fir_hbm.at[pl.ds(row0, CH)])

    return k(fs)


def _combine_body(C, fir_ref, xe_ref, cons_ref, out_ref, nrm_ref):
    fir = fir_ref[...]
    denom = jnp.sum(fir, axis=1, keepdims=True) + 1e-9
    nrm = fir / denom
    nrm_ref[...] = nrm
    cons2 = cons_ref[...]
    nj = cons2.shape[1] // C
    csum = cons2[:, 0:C]
    for j in range(1, nj):
        csum = csum + cons2[:, j * C:(j + 1) * C]
    w = lax.dot_general(nrm, csum, (((1,), (0,)), ((), ())),
                        preferred_element_type=jnp.float32)
    sx = jnp.sum(xe_ref[...], axis=1, keepdims=True)
    out_ref[...] = sx * w


def _combine(fir, x, consequents):
    B, R = fir.shape
    D = x.shape[1]
    C = consequents.shape[2]
    xe = jnp.concatenate([x, jnp.ones((B, 1), x.dtype)], axis=1)
    cons = consequents.reshape(R, (D + 1) * C)
    return pl.pallas_call(
        functools.partial(_combine_body, C),
        in_specs=[
            pl.BlockSpec((B, R), lambda: (0, 0)),
            pl.BlockSpec((B, D + 1), lambda: (0, 0)),
            pl.BlockSpec((R, (D + 1) * C), lambda: (0, 0)),
        ],
        out_specs=[
            pl.BlockSpec((B, C), lambda: (0, 0)),
            pl.BlockSpec((B, R), lambda: (0, 0)),
        ],
        out_shape=[
            jax.ShapeDtypeStruct((B, C), jnp.float32),
            jax.ShapeDtypeStruct((B, R), jnp.float32),
        ],
    )(fir, xe, cons)


def kernel(x, centers, widths, consequents, rules):
    R = rules.shape[0]
    K = max(1, int(0.2 * R))
    fs = _firing_strengths(x, centers, widths, rules)
    firing, mask = _sc_topk_mask(fs, K)
    rule_outputs, normalized = _combine(firing, x, consequents)
    return (rule_outputs, normalized, mask)
